# baseline (speedup 1.0000x reference)
import jax
import jax.numpy as jnp
import numpy as np
from jax.experimental import pallas as pl

NUM_STAGES = 3
EMBED_DIM = 96
K_NEIGHBORS = 32
ALPHA = 1000.0
BETA = 100.0


def _index_points(pts, idx):
    return jax.vmap(lambda p, i: p[i])(pts, idx)


def _fps(xyz, n_group):
    B, N, _ = xyz.shape

    def body(i, state):
        idx, dists, farthest = state
        idx = idx.at[:, i].set(farthest)
        centroid = jax.vmap(lambda p, f: p[f])(xyz, farthest)
        d = jnp.sum((xyz - centroid[:, None, :]) ** 2, axis=-1)
        dists = jnp.minimum(dists, d)
        farthest = jnp.argmax(dists, axis=-1).astype(jnp.int32)
        return idx, dists, farthest

    idx0 = jnp.zeros((B, n_group), dtype=jnp.int32)
    d0 = jnp.full((B, N), 1e10, dtype=jnp.float32)
    f0 = jnp.zeros((B,), dtype=jnp.int32)
    idx, _, _ = jax.lax.fori_loop(0, n_group, body, (idx0, d0, f0))
    return idx


def _knn_idx(k, xyz, centers):
    d2 = jnp.sum((centers[:, :, None, :] - xyz[:, None, :, :]) ** 2, axis=-1)
    _, idx = jax.lax.top_k(-d2, k)
    return idx


def _normalize(x, ref):
    mean = ref[:, :, None, :]
    std = jnp.std(x - mean, ddof=1)
    return (x - mean) / (std + 1e-05)


def _fourier(x4, Bmat):
    xp = jnp.transpose(x4, (0, 2, 3, 1))
    proj = xp @ Bmat
    ff = jnp.concatenate([jnp.sin(2.0 * jnp.pi * proj), jnp.cos(2.0 * jnp.pi * proj)], axis=-1)
    ff = jnp.sign(ff) * jnp.abs(ff) ** 5
    return jnp.transpose(ff, (0, 3, 1, 2))


def _tr_embed(my_dim, T):
    B, in_dim, G, K = T.shape
    feat_dim = my_dim // (2 * in_dim)
    fr = jnp.arange(feat_dim, dtype=jnp.float32)
    dim_embed = jnp.power(ALPHA, fr / feat_dim)
    div = BETA * T[..., None] / dim_embed
    s = jnp.sin(div)
    c = jnp.cos(div)
    e = jnp.stack([s, c], axis=5).reshape(B, in_dim, G, K, 2 * feat_dim)
    e = jnp.transpose(e, (0, 1, 4, 2, 3)).reshape(B, my_dim, G, K)
    return e


def _lga(lc_xyz, lc_x, knn_xyz, knn_x, Bmat):
    B, G, K, C = knn_x.shape
    knn_x = _normalize(knn_x, lc_x)
    knn_xyz = _normalize(knn_xyz, lc_xyz)
    lc = jnp.broadcast_to(lc_xyz[:, :, None, :], (B, G, K, 3))
    lc = jnp.transpose(lc, (0, 3, 1, 2))
    kx = jnp.transpose(knn_x, (0, 3, 1, 2))
    kz = jnp.transpose(knn_xyz, (0, 3, 1, 2))
    cross = jnp.cross(kz, lc, axis=1)
    dot = jnp.sum(kz * lc, axis=1, keepdims=True)
    emb = jnp.concatenate([kz, cross, dot], axis=1)
    emb = _fourier(emb, Bmat)
    knn_x_w = jnp.concatenate([kx, emb], axis=1)
    pos = _tr_embed(2 * C, kz) + _tr_embed(2 * C, lc)
    final_x = knn_x_w + pos
    final_x = final_x * pos
    return final_x


def _identity_kernel(x_ref, o_ref):
    o_ref[...] = x_ref[...]


def kernel(xyz, x):
    for s in range(NUM_STAGES):
        B, N, _ = xyz.shape
        G = N // 2
        fps_idx = _fps(jax.lax.stop_gradient(xyz), G)
        lc_xyz = _index_points(xyz, fps_idx)
        lc_x = _index_points(x, fps_idx)
        knn_idx = _knn_idx(K_NEIGHBORS, xyz, lc_xyz)
        knn_xyz = _index_points(xyz, knn_idx)
        knn_x = _index_points(x, knn_idx)
        C = x.shape[-1]
        Bmat = jnp.asarray(np.random.default_rng(100 + s).standard_normal((7, C // 2)).astype(np.float32) * 50.0)
        f = _lga(lc_xyz, lc_x, knn_xyz, knn_x, Bmat)
        pooled = jnp.max(f, axis=-1) + jnp.mean(f, axis=-1)
        x = jnp.transpose(pooled, (0, 2, 1))
        xyz = lc_xyz
    out = jnp.transpose(x, (0, 2, 1))
    return pl.pallas_call(
        _identity_kernel,
        out_shape=jax.ShapeDtypeStruct(out.shape, out.dtype),
    )(out)


# speedup vs baseline: 3.2370x; 3.2370x over previous
import functools

import jax
import jax.numpy as jnp
import numpy as np
from jax.experimental import pallas as pl

NUM_STAGES = 3
K = 32
ALPHA = 1000.0
BETA = 100.0
TWO_PI = 2.0 * np.pi
GBLK1 = 64


def _fps_kernel(xyzT_ref, lcT_ref, *, G, N, B):
    xyz = xyzT_ref[...]
    lane_n = jax.lax.broadcasted_iota(jnp.int32, (B, N), 1)
    lane_g = jax.lax.broadcasted_iota(jnp.int32, (B, 3, G), 2)
    lcT_ref[...] = jnp.zeros((B, 3, G), jnp.float32)

    def body(i, state):
        dists, far = state
        maskf = (lane_n == far).astype(jnp.float32)[:, None, :]
        cent = jnp.sum(xyz * maskf, axis=2, keepdims=True)
        mgf = (lane_g == i).astype(jnp.float32)
        lcT_ref[...] = lcT_ref[...] + mgf * cent
        diff = xyz - cent
        sq = diff * diff
        d = (sq[:, 0, :] + sq[:, 1, :]) + sq[:, 2, :]
        dists = jnp.minimum(dists, d)
        m = jnp.max(dists, axis=1, keepdims=True)
        not_m = (dists != m).astype(jnp.int32)
        far = jnp.min(lane_n + not_m * N, axis=1, keepdims=True)
        return dists, far.astype(jnp.int32)

    d0 = jnp.full((B, N), 1e10, dtype=jnp.float32)
    f0 = jnp.zeros((B, 1), dtype=jnp.int32)
    jax.lax.fori_loop(0, G, body, (d0, f0))


def _knn_kernel(lc_ref, xyzT_ref, xyz_ref, x_ref,
                v_ref, vz_ref, sx_ref, sxx_ref, sz_ref, szz_ref, *, N, C):
    lc = lc_ref[0]
    xyzT = xyzT_ref[0]
    xyz = xyz_ref[0]
    xmat = x_ref[0]
    iota = jax.lax.broadcasted_iota(jnp.int32, (GBLK1, N), 1)

    s0 = (lc[:, 0:1] - xyzT[0:1, :]) ** 2
    s1 = (lc[:, 1:2] - xyzT[1:2, :]) ** 2
    s2 = (lc[:, 2:3] - xyzT[2:3, :]) ** 2
    cur = (s0 + s1) + s2

    sx = jnp.float32(0.0)
    sxx = jnp.float32(0.0)
    sz = jnp.float32(0.0)
    szz = jnp.float32(0.0)
    xg0 = None
    for j in range(K):
        m = jnp.min(cur, axis=1, keepdims=True)
        not_m = (cur != m).astype(jnp.int32)
        sel = jnp.min(iota + not_m * N, axis=1, keepdims=True)
        maskf = (iota == sel).astype(jnp.float32)
        xg = jnp.dot(maskf, xmat, precision=jax.lax.Precision.HIGHEST)
        zg = jnp.dot(maskf, xyz, precision=jax.lax.Precision.HIGHEST)
        if j == 0:
            xg0 = xg
        vj = xg - xg0
        vzj = zg - lc
        v_ref[0, :, j, :] = vj
        vz_ref[0, :, j, :] = vzj
        sx = sx + jnp.sum(vj)
        sxx = sxx + jnp.sum(vj * vj)
        sz = sz + jnp.sum(vzj)
        szz = szz + jnp.sum(vzj * vzj)
        if j < K - 1:
            cur = cur + maskf * jnp.float32(1e30)
    sx_ref[0, 0] = jnp.reshape(sx, (1, 1))
    sxx_ref[0, 0] = jnp.reshape(sxx, (1, 1))
    sz_ref[0, 0] = jnp.reshape(sz, (1, 1))
    szz_ref[0, 0] = jnp.reshape(szz, (1, 1))


def _feat_kernel(v_ref, vz_ref, lc_ref, stdx_ref, stdz_ref, bmat_ref,
                 isel_ref, scoff_ref, denom_ref, out_ref, *, C, GBLK2):
    C2 = 2 * C
    GK = GBLK2 * K
    kx = v_ref[0].reshape(GK, C) / (stdx_ref[0, 0] + 1e-5)
    kz = vz_ref[0].reshape(GK, 3) / (stdz_ref[0, 0] + 1e-5)
    lc3 = jnp.broadcast_to(lc_ref[0][:, None, :], (GBLK2, K, 3)).reshape(GK, 3)

    kz0, kz1, kz2 = kz[:, 0:1], kz[:, 1:2], kz[:, 2:3]
    l0, l1, l2 = lc3[:, 0:1], lc3[:, 1:2], lc3[:, 2:3]
    cr0 = kz1 * l2 - kz2 * l1
    cr1 = kz2 * l0 - kz0 * l2
    cr2 = kz0 * l1 - kz1 * l0
    dt = (kz0 * l0 + kz1 * l1) + kz2 * l2
    emb7 = jnp.concatenate([kz0, kz1, kz2, cr0, cr1, cr2, dt], axis=1)

    proj = jnp.dot(emb7, bmat_ref[...])
    arg = TWO_PI * proj
    ff = jnp.concatenate([jnp.sin(arg), jnp.cos(arg)], axis=1)
    p2 = ff * ff
    ff5 = ff * (p2 * p2)

    isel = isel_ref[...]
    scoff = scoff_ref[...]
    denom = denom_ref[...]

    def pos_of(t):
        t0, t1, t2 = t[:, 0:1], t[:, 1:2], t[:, 2:3]
        tl = jnp.where(isel == 0, t0, jnp.where(isel == 1, t1, t2))
        return jnp.sin((BETA * tl) / denom + scoff)

    pos = pos_of(kz) + pos_of(lc3)
    w = jnp.concatenate([kx, ff5], axis=1)
    fin = (w + pos) * pos
    f3 = fin.reshape(GBLK2, K, C2)
    out_ref[0] = jnp.max(f3, axis=1) + jnp.sum(f3, axis=1) * (1.0 / K)


def _stage(xyz, xyzT, x, s):
    B, N, _ = xyz.shape
    G = N // 2
    C = x.shape[-1]
    C2 = 2 * C
    nblk1 = G // GBLK1

    lcT = pl.pallas_call(
        functools.partial(_fps_kernel, G=G, N=N, B=B),
        out_shape=jax.ShapeDtypeStruct((B, 3, G), jnp.float32),
    )(xyzT)
    lc_cols = jnp.transpose(lcT, (0, 2, 1))

    sum_shape = jax.ShapeDtypeStruct((B, nblk1, 1, 1), jnp.float32)
    sum_spec = pl.BlockSpec((1, 1, 1, 1), lambda b, j: (b, j, 0, 0))
    v, vz, sx, sxx, sz, szz = pl.pallas_call(
        functools.partial(_knn_kernel, N=N, C=C),
        grid=(B, nblk1),
        in_specs=[
            pl.BlockSpec((1, GBLK1, 3), lambda b, j: (b, j, 0)),
            pl.BlockSpec((1, 3, N), lambda b, j: (b, 0, 0)),
            pl.BlockSpec((1, N, 3), lambda b, j: (b, 0, 0)),
            pl.BlockSpec((1, N, C), lambda b, j: (b, 0, 0)),
        ],
        out_shape=[
            jax.ShapeDtypeStruct((B, G, K, C), jnp.float32),
            jax.ShapeDtypeStruct((B, G, K, 3), jnp.float32),
            sum_shape, sum_shape, sum_shape, sum_shape,
        ],
        out_specs=[
            pl.BlockSpec((1, GBLK1, K, C), lambda b, j: (b, j, 0, 0)),
            pl.BlockSpec((1, GBLK1, K, 3), lambda b, j: (b, j, 0, 0)),
            sum_spec, sum_spec, sum_spec, sum_spec,
        ],
    )(lc_cols, xyzT, xyz, x)

    n_x = B * G * K * C
    n_z = B * G * K * 3
    tx = jnp.sum(sx)
    tz = jnp.sum(sz)
    var_x = jnp.maximum((jnp.sum(sxx) - tx * tx / n_x) / (n_x - 1), 0.0)
    var_z = jnp.maximum((jnp.sum(szz) - tz * tz / n_z) / (n_z - 1), 0.0)
    std_x = jnp.sqrt(var_x).reshape(1, 1)
    std_z = jnp.sqrt(var_z).reshape(1, 1)

    bmat = jnp.asarray(
        np.random.default_rng(100 + s).standard_normal((7, C // 2)).astype(np.float32) * 50.0)
    fd = C2 // 6
    cc = np.arange(C2)
    isel = jnp.asarray((cc // (2 * fd)).astype(np.int32)[None, :])
    scoff = jnp.asarray(np.where(cc % 2 == 0, 0.0, np.pi / 2).astype(np.float32)[None, :])
    f_idx = ((cc % (2 * fd)) // 2).astype(np.float32)
    denom = jnp.asarray(
        np.power(np.float32(ALPHA), f_idx / np.float32(fd)).astype(np.float32)[None, :])

    GBLK2 = max(8, 6144 // C)
    nblk2 = G // GBLK2
    const_spec2 = pl.BlockSpec((1, C2), lambda b, j: (0, 0))
    xn = pl.pallas_call(
        functools.partial(_feat_kernel, C=C, GBLK2=GBLK2),
        grid=(B, nblk2),
        in_specs=[
            pl.BlockSpec((1, GBLK2, K, C), lambda b, j: (b, j, 0, 0)),
            pl.BlockSpec((1, GBLK2, K, 3), lambda b, j: (b, j, 0, 0)),
            pl.BlockSpec((1, GBLK2, 3), lambda b, j: (b, j, 0)),
            pl.BlockSpec((1, 1), lambda b, j: (0, 0)),
            pl.BlockSpec((1, 1), lambda b, j: (0, 0)),
            pl.BlockSpec((7, C // 2), lambda b, j: (0, 0)),
            const_spec2, const_spec2, const_spec2,
        ],
        out_shape=jax.ShapeDtypeStruct((B, G, C2), jnp.float32),
        out_specs=pl.BlockSpec((1, GBLK2, C2), lambda b, j: (b, j, 0)),
    )(v, vz, lc_cols, std_x, std_z, bmat, isel, scoff, denom)

    return lc_cols, lcT, xn


def kernel(xyz, x):
    xyzT = jnp.transpose(xyz, (0, 2, 1))
    for s in range(NUM_STAGES):
        xyz, xyzT, x = _stage(xyz, xyzT, x, s)
    return jnp.transpose(x, (0, 2, 1))


# speedup vs baseline: 4.9590x; 1.5320x over previous
import functools

import jax
import jax.numpy as jnp
import numpy as np
from jax.experimental import pallas as pl

NUM_STAGES = 3
K = 32
ALPHA = 1000.0
BETA = 100.0
TWO_PI = 2.0 * np.pi
GBLK1 = 64

_SC0 = np.float32(9.99999972e-01)
_SC1 = np.float32(-1.66666443e-01)
_SC2 = np.float32(8.33283921e-03)
_SC3 = np.float32(-1.97969742e-04)
_SC4 = np.float32(2.58228439e-06)
_P1 = np.float32(3.140625)
_P2 = np.float32(0.0009651184)
_P3 = np.float32(2.5331974e-06)
_P4 = np.float32(1.984187e-09)
_INV_PI = np.float32(1.0 / np.pi)
_PI_HALF = np.float32(np.pi / 2.0)


def _fast_sin(x, thalf=None):
    nf = x * _INV_PI if thalf is None else x * _INV_PI + 0.5 * thalf
    n = jnp.round(nf)
    r = (((x - n * _P1) - n * _P2) - n * _P3) - n * _P4
    if thalf is not None:
        r = r + thalf * _PI_HALF
    r2 = r * r
    p = _SC0 + r2 * (_SC1 + r2 * (_SC2 + r2 * (_SC3 + r2 * _SC4)))
    h = n * 0.5
    sgn = 1.0 - 4.0 * (h - jnp.floor(h))
    return (r * p) * sgn


def _fps_kernel(xyzT_ref, lcT_ref, *, G, N, B):
    xyz = xyzT_ref[...]
    lane_n = jax.lax.broadcasted_iota(jnp.int32, (B, N), 1)
    lane_g = jax.lax.broadcasted_iota(jnp.int32, (B, 3, G), 2)
    lcT_ref[...] = jnp.zeros((B, 3, G), jnp.float32)

    def body(i, state):
        dists, far = state
        maskf = (lane_n == far).astype(jnp.float32)[:, None, :]
        cent = jnp.sum(xyz * maskf, axis=2, keepdims=True)
        mgf = (lane_g == i).astype(jnp.float32)
        lcT_ref[...] = lcT_ref[...] + mgf * cent
        diff = xyz - cent
        sq = diff * diff
        d = (sq[:, 0, :] + sq[:, 1, :]) + sq[:, 2, :]
        dists = jnp.minimum(dists, d)
        m = jnp.max(dists, axis=1, keepdims=True)
        not_m = (dists != m).astype(jnp.int32)
        far = jnp.min(lane_n + not_m * N, axis=1, keepdims=True)
        return dists, far.astype(jnp.int32)

    d0 = jnp.full((B, N), 1e10, dtype=jnp.float32)
    f0 = jnp.zeros((B, 1), dtype=jnp.int32)
    jax.lax.fori_loop(0, G, body, (d0, f0))


def _knn_kernel(lc_ref, xyzT_ref, hi_ref, mid_ref, lo_ref,
                v_ref, vz_ref, sx_ref, sxx_ref, sz_ref, szz_ref, *, N, C):
    lc = lc_ref[0]
    xyzT = xyzT_ref[0]
    iota = jax.lax.broadcasted_iota(jnp.int32, (GBLK1, N), 1)

    s0 = (lc[:, 0:1] - xyzT[0:1, :]) ** 2
    s1 = (lc[:, 1:2] - xyzT[1:2, :]) ** 2
    s2 = (lc[:, 2:3] - xyzT[2:3, :]) ** 2
    cur = (s0 + s1) + s2

    sels = []
    for j in range(K):
        m = jnp.min(cur, axis=1, keepdims=True)
        not_m = (cur != m).astype(jnp.int32)
        sel = jnp.min(iota + not_m * N, axis=1, keepdims=True)
        sels.append(sel)
        if j < K - 1:
            cur = cur + (iota == sel).astype(jnp.float32) * jnp.float32(1e30)
    sel_mat = jnp.concatenate(sels, axis=1)
    iota3 = jax.lax.broadcasted_iota(jnp.int32, (GBLK1, K, N), 2)
    onehot = (sel_mat[:, :, None] == iota3).astype(jnp.bfloat16).reshape(GBLK1 * K, N)
    f32 = jnp.float32
    g = (jnp.dot(onehot, hi_ref[0], preferred_element_type=f32)
         + jnp.dot(onehot, mid_ref[0], preferred_element_type=f32)) \
        + jnp.dot(onehot, lo_ref[0], preferred_element_type=f32)
    g3 = g.reshape(GBLK1, K, C + 3)
    v3 = g3[:, :, :C] - g3[:, 0:1, :C]
    vz3 = g3[:, :, C:] - lc[:, None, :]
    v_ref[0] = v3
    vz_ref[0] = vz3
    sx_ref[0, 0] = jnp.reshape(jnp.sum(v3), (1, 1))
    sxx_ref[0, 0] = jnp.reshape(jnp.sum(v3 * v3), (1, 1))
    sz_ref[0, 0] = jnp.reshape(jnp.sum(vz3), (1, 1))
    szz_ref[0, 0] = jnp.reshape(jnp.sum(vz3 * vz3), (1, 1))


def _feat_kernel(v_ref, vz_ref, lc_ref, stdx_ref, stdz_ref, bmat_ref,
                 isel_ref, scoff_ref, invden_ref, out_ref, *, C, GBLK2):
    C2 = 2 * C
    GK = GBLK2 * K
    rx = 1.0 / (stdx_ref[0, 0] + 1e-5)
    rz = 1.0 / (stdz_ref[0, 0] + 1e-5)
    kx3 = v_ref[0] * rx
    kz = vz_ref[0].reshape(GK, 3) * rz
    lc2 = lc_ref[0]
    lc3 = jnp.broadcast_to(lc_ref[0][:, None, :], (GBLK2, K, 3)).reshape(GK, 3)

    kz0, kz1, kz2 = kz[:, 0:1], kz[:, 1:2], kz[:, 2:3]
    l0, l1, l2 = lc3[:, 0:1], lc3[:, 1:2], lc3[:, 2:3]
    cr0 = kz1 * l2 - kz2 * l1
    cr1 = kz2 * l0 - kz0 * l2
    cr2 = kz0 * l1 - kz1 * l0
    dt = (kz0 * l0 + kz1 * l1) + kz2 * l2
    emb7 = jnp.concatenate([kz0, kz1, kz2, cr0, cr1, cr2, dt], axis=1)

    proj = jnp.dot(emb7, bmat_ref[...])
    arg = TWO_PI * proj
    ff = jnp.concatenate([_fast_sin(arg), _fast_sin(arg, 1.0)], axis=1)
    p2 = ff * ff
    ff53 = (ff * (p2 * p2)).reshape(GBLK2, K, C)

    isel = isel_ref[...]
    scoff = scoff_ref[...]
    invden = invden_ref[...]

    def pos_of(t):
        t0, t1, t2 = t[:, 0:1], t[:, 1:2], t[:, 2:3]
        tl = jnp.where(isel == 0, t0, jnp.where(isel == 1, t1, t2))
        return _fast_sin((BETA * tl) * invden, scoff)

    pos = pos_of(kz).reshape(GBLK2, K, C2) + pos_of(lc2)[:, None, :]
    w = jnp.concatenate([kx3, ff53], axis=2)
    fin = (w + pos) * pos
    out_ref[0] = jnp.max(fin, axis=1) + jnp.sum(fin, axis=1) * (1.0 / K)


def _stage(xyz, xyzT, x, s):
    B, N, _ = xyz.shape
    G = N // 2
    C = x.shape[-1]
    C2 = 2 * C
    nblk1 = G // GBLK1

    lcT = pl.pallas_call(
        functools.partial(_fps_kernel, G=G, N=N, B=B),
        out_shape=jax.ShapeDtypeStruct((B, 3, G), jnp.float32),
    )(xyzT)
    lc_cols = jnp.transpose(lcT, (0, 2, 1))

    src = jnp.concatenate([x, xyz], axis=-1)
    s_hi = src.astype(jnp.bfloat16)
    r1 = src - s_hi.astype(jnp.float32)
    s_mid = r1.astype(jnp.bfloat16)
    s_lo = (r1 - s_mid.astype(jnp.float32)).astype(jnp.bfloat16)
    src_spec = pl.BlockSpec((1, N, C + 3), lambda b, j: (b, 0, 0))
    sum_shape = jax.ShapeDtypeStruct((B, nblk1, 1, 1), jnp.float32)
    sum_spec = pl.BlockSpec((1, 1, 1, 1), lambda b, j: (b, j, 0, 0))
    v, vz, sx, sxx, sz, szz = pl.pallas_call(
        functools.partial(_knn_kernel, N=N, C=C),
        grid=(B, nblk1),
        in_specs=[
            pl.BlockSpec((1, GBLK1, 3), lambda b, j: (b, j, 0)),
            pl.BlockSpec((1, 3, N), lambda b, j: (b, 0, 0)),
            src_spec, src_spec, src_spec,
        ],
        out_shape=[
            jax.ShapeDtypeStruct((B, G, K, C), jnp.float32),
            jax.ShapeDtypeStruct((B, G, K, 3), jnp.float32),
            sum_shape, sum_shape, sum_shape, sum_shape,
        ],
        out_specs=[
            pl.BlockSpec((1, GBLK1, K, C), lambda b, j: (b, j, 0, 0)),
            pl.BlockSpec((1, GBLK1, K, 3), lambda b, j: (b, j, 0, 0)),
            sum_spec, sum_spec, sum_spec, sum_spec,
        ],
    )(lc_cols, xyzT, s_hi, s_mid, s_lo)

    n_x = B * G * K * C
    n_z = B * G * K * 3
    tx = jnp.sum(sx)
    tz = jnp.sum(sz)
    var_x = jnp.maximum((jnp.sum(sxx) - tx * tx / n_x) / (n_x - 1), 0.0)
    var_z = jnp.maximum((jnp.sum(szz) - tz * tz / n_z) / (n_z - 1), 0.0)
    std_x = jnp.sqrt(var_x).reshape(1, 1)
    std_z = jnp.sqrt(var_z).reshape(1, 1)

    bmat = jnp.asarray(
        np.random.default_rng(100 + s).standard_normal((7, C // 2)).astype(np.float32) * 50.0)
    fd = C2 // 6
    cc = np.arange(C2)
    isel = jnp.asarray((cc // (2 * fd)).astype(np.int32)[None, :])
    scoff = jnp.asarray(np.where(cc % 2 == 0, 0.0, 1.0).astype(np.float32)[None, :])
    f_idx = ((cc % (2 * fd)) // 2).astype(np.float32)
    invden = jnp.asarray(
        (1.0 / np.power(np.float32(ALPHA), f_idx / np.float32(fd))).astype(np.float32)[None, :])

    GBLK2 = max(8, 6144 // C)
    nblk2 = G // GBLK2
    const_spec2 = pl.BlockSpec((1, C2), lambda b, j: (0, 0))
    xn = pl.pallas_call(
        functools.partial(_feat_kernel, C=C, GBLK2=GBLK2),
        grid=(B, nblk2),
        in_specs=[
            pl.BlockSpec((1, GBLK2, K, C), lambda b, j: (b, j, 0, 0)),
            pl.BlockSpec((1, GBLK2, K, 3), lambda b, j: (b, j, 0, 0)),
            pl.BlockSpec((1, GBLK2, 3), lambda b, j: (b, j, 0)),
            pl.BlockSpec((1, 1), lambda b, j: (0, 0)),
            pl.BlockSpec((1, 1), lambda b, j: (0, 0)),
            pl.BlockSpec((7, C // 2), lambda b, j: (0, 0)),
            const_spec2, const_spec2, const_spec2,
        ],
        out_shape=jax.ShapeDtypeStruct((B, G, C2), jnp.float32),
        out_specs=pl.BlockSpec((1, GBLK2, C2), lambda b, j: (b, j, 0)),
    )(v, vz, lc_cols, std_x, std_z, bmat, isel, scoff, invden)

    return lc_cols, lcT, xn


def kernel(xyz, x):
    xyzT = jnp.transpose(xyz, (0, 2, 1))
    for s in range(NUM_STAGES):
        xyz, xyzT, x = _stage(xyz, xyzT, x, s)
    return jnp.transpose(x, (0, 2, 1))
